# Initial kernel scaffold; baseline (speedup 1.0000x reference)
import jax
import jax.numpy as jnp
from jax.experimental import pallas as pl


def kernel(input_ids, token_type_ids, word_emb, token_type_emb, pos_emb, ln_weight, ln_bias):
    raise NotImplementedError("write your pallas kernel here")


# speedup vs baseline: 1.5316x; 1.5316x over previous
import functools

import jax
import jax.numpy as jnp
from jax import lax
from jax.experimental import pallas as pl
from jax.experimental.pallas import tpu as pltpu
from jax.experimental.pallas import tpu_sc as plsc

EPS = 1e-12

_NC = 2
_NS = 16
_NW = _NC * _NS

_CHUNK = 64


def _sc_gather(table, ids):
    n_tok = ids.shape[0]
    hidden = table.shape[1]
    per_w = n_tok // _NW
    n_chunks = per_w // _CHUNK

    mesh = plsc.VectorSubcoreMesh(core_axis_name="c", subcore_axis_name="s")

    @functools.partial(
        pl.kernel,
        mesh=mesh,
        out_type=jax.ShapeDtypeStruct((n_tok, hidden), jnp.float32),
        scratch_types=[
            pltpu.VMEM((per_w,), jnp.int32),
            pltpu.VMEM((_CHUNK, hidden), jnp.float32),
            pltpu.VMEM((_CHUNK, hidden), jnp.float32),
            pltpu.SemaphoreType.DMA,
            pltpu.SemaphoreType.DMA,
        ],
    )
    def gather_kernel(table_hbm, ids_hbm, out_hbm, idx_v, buf0, buf1, sem0, sem1):
        wid = lax.axis_index("s") * _NC + lax.axis_index("c")
        base = wid * per_w
        pltpu.sync_copy(ids_hbm.at[pl.ds(base, per_w)], idx_v)
        bufs = (buf0, buf1)
        sems = (sem0, sem1)
        copies = [None] * n_chunks
        copies[0] = pltpu.async_copy(
            table_hbm.at[idx_v.at[pl.ds(0, _CHUNK)]], buf0, sem0
        )
        for k in range(n_chunks):
            nxt = k + 1
            if nxt < n_chunks:
                copies[nxt] = pltpu.async_copy(
                    table_hbm.at[idx_v.at[pl.ds(nxt * _CHUNK, _CHUNK)]],
                    bufs[nxt % 2],
                    sems[nxt % 2],
                )
            copies[k].wait()
            pltpu.sync_copy(bufs[k % 2], out_hbm.at[pl.ds(base + k * _CHUNK, _CHUNK)])

    return gather_kernel(table, ids)


def _tc_add_ln(gathered, tt_ids, pos_emb, tt_emb, ln_w, ln_b, seq):
    n_tok, hidden = gathered.shape
    tb = 256
    n_blocks = n_tok // tb
    pos_blocks = seq // tb

    tt3 = tt_ids.reshape(n_blocks, 1, tb)

    def body(g_ref, tt_ref, pos_ref, tte_ref, w_ref, b_ref, o_ref):
        x = g_ref[...] + pos_ref[...]
        ttf = tt_ref[0, 0, :].astype(jnp.float32)
        t0 = tte_ref[0, :]
        t1 = tte_ref[1, :]
        x = x + t0[None, :] + ttf[:, None] * (t1 - t0)[None, :]
        u = jnp.mean(x, axis=-1, keepdims=True)
        s = jnp.mean((x - u) ** 2, axis=-1, keepdims=True)
        y = (x - u) * lax.rsqrt(s + EPS)
        o_ref[...] = y * w_ref[0, :][None, :] + b_ref[0, :][None, :]

    return pl.pallas_call(
        body,
        grid=(n_blocks,),
        in_specs=[
            pl.BlockSpec((tb, hidden), lambda i: (i, 0)),
            pl.BlockSpec((1, 1, tb), lambda i: (i, 0, 0)),
            pl.BlockSpec((tb, hidden), lambda i: (i % pos_blocks, 0)),
            pl.BlockSpec((2, hidden), lambda i: (0, 0)),
            pl.BlockSpec((1, hidden), lambda i: (0, 0)),
            pl.BlockSpec((1, hidden), lambda i: (0, 0)),
        ],
        out_specs=pl.BlockSpec((tb, hidden), lambda i: (i, 0)),
        out_shape=jax.ShapeDtypeStruct((n_tok, hidden), jnp.float32),
    )(gathered, tt3, pos_emb, tt_emb, ln_w.reshape(1, hidden), ln_b.reshape(1, hidden))


def kernel(input_ids, token_type_ids, word_emb, token_type_emb, pos_emb, ln_weight, ln_bias):
    batch, seq = input_ids.shape
    hidden = word_emb.shape[1]
    ids = input_ids.reshape(-1).astype(jnp.int32)
    tt_ids = token_type_ids.reshape(-1).astype(jnp.int32)
    gathered = _sc_gather(word_emb, ids)
    out = _tc_add_ln(gathered, tt_ids, pos_emb, token_type_emb, ln_weight, ln_bias, seq)
    return out.reshape(batch, seq, hidden)
